# Initial kernel scaffold; baseline (speedup 1.0000x reference)
import jax
import jax.numpy as jnp
from jax.experimental import pallas as pl


def kernel(cls_input, center_rate):
    raise NotImplementedError("write your pallas kernel here")


# speedup vs baseline: 6.9177x; 6.9177x over previous
import jax
import jax.numpy as jnp
from jax import lax
from jax.experimental import pallas as pl
from jax.experimental.pallas import tpu as pltpu

_B = 64
_R = 2048
_L = 128
_N = _R * _L
_TOPK = 128
_INV_DEN = 1.0 / 511.0
_SCALE = 1.0 / (_B * _TOPK * 2)


def _row_body(x_ref, cr_ref, out_ref, sb_ref):
    i = pl.program_id(0)

    s = 1.0 / (1.0 + jnp.exp(-x_ref[0]))
    sb = lax.bitcast_convert_type(s, jnp.int32)
    sb_ref[...] = sb

    lane_max = jnp.max(sb, axis=0)
    lo0 = jnp.min(lane_max)
    hi0 = jnp.max(lane_max) + 1

    def bs_cond(c):
        lo, hi = c
        return hi - lo > 1

    def bs_body(c):
        lo, hi = c
        mid = lo + (hi - lo) // 2
        cnt = jnp.sum((sb_ref[...] >= mid).astype(jnp.int32))
        ge = cnt >= _TOPK
        return (jnp.where(ge, mid, lo), jnp.where(ge, hi, mid))

    tstar, _ = lax.while_loop(bs_cond, bs_body, (lo0, hi0))

    sbv = sb_ref[...]
    m_gt = sbv > tstar
    m_eq = sbv == tstar
    cnt_gt = jnp.sum(m_gt.astype(jnp.int32))
    cnt_eq = jnp.sum(m_eq.astype(jnp.int32))
    r = _TOPK - cnt_gt

    rr = lax.broadcasted_iota(jnp.int32, (_R, _L), 0)
    cc = lax.broadcasted_iota(jnp.int32, (_R, _L), 1)
    k = rr * _L + cc
    p0 = (k >> 9).astype(jnp.float32) * _INV_DEN
    p1 = (k & 511).astype(jnp.float32) * _INV_DEN
    q = 0.5 * (p0 * p0 + p1 * p1)

    zf = jnp.float32(0.0)

    def masked_sums(m):
        return (jnp.sum(jnp.where(m, p0, zf)),
                jnp.sum(jnp.where(m, p1, zf)),
                jnp.sum(jnp.where(m, q, zf)))

    s0_gt, s1_gt, sq_gt = masked_sums(m_gt)

    def eq_all(_):
        return masked_sums(m_eq)

    def eq_partial(_):
        def j_cond(c):
            lo_j, hi_j = c
            return hi_j - lo_j > 1

        def j_body(c):
            lo_j, hi_j = c
            mid = lo_j + (hi_j - lo_j) // 2
            c_eq = jnp.sum((m_eq & (k < mid)).astype(jnp.int32))
            ge = c_eq >= r
            return (jnp.where(ge, lo_j, mid), jnp.where(ge, mid, hi_j))

        _, hi_j = lax.while_loop(j_cond, j_body,
                                 (jnp.int32(0), jnp.int32(_N)))
        cut = jnp.where(r > 0, hi_j, 0)
        return masked_sums(m_eq & (k < cut))

    s0_eq, s1_eq, sq_eq = lax.cond(cnt_eq == r, eq_all, eq_partial, 0)

    S0 = s0_gt + s0_eq
    S1 = s1_gt + s1_eq
    Sq = sq_gt + sq_eq

    c0 = cr_ref[0, 0:1, :]
    c1 = cr_ref[0, 1:2, :]
    contrib = (Sq + (_TOPK * 0.5) * (c0 * c0 + c1 * c1)
               - c0 * S0 - c1 * S1) * _SCALE

    @pl.when(i == 0)
    def _init():
        out_ref[...] = jnp.zeros_like(out_ref)

    out_ref[...] += contrib


def kernel(cls_input, center_rate):
    x3 = cls_input.reshape(_B, _R, _L)
    cr3 = center_rate.T.reshape(_B, 2, 1)
    out = pl.pallas_call(
        _row_body,
        grid=(_B,),
        in_specs=[
            pl.BlockSpec((1, _R, _L), lambda i: (i, 0, 0)),
            pl.BlockSpec((1, 2, 1), lambda i: (i, 0, 0)),
        ],
        out_specs=pl.BlockSpec((1, 1), lambda i: (0, 0)),
        out_shape=jax.ShapeDtypeStruct((1, 1), jnp.float32),
        scratch_shapes=[pltpu.VMEM((_R, _L), jnp.int32)],
    )(x3, cr3)
    return out[0, 0]
